# baseline (speedup 1.0000x reference)
import functools

import jax
import jax.numpy as jnp
from jax import lax
from jax.experimental import pallas as pl
from jax.experimental.pallas import tpu as pltpu
from jax.experimental.pallas import tpu_sc as plsc

D_MODEL = 64
D_PAD = 128
NUM_CORES = 2
NUM_SUBCORES = 16
NUM_WORKERS = NUM_CORES * NUM_SUBCORES
NBUF = 2
VB = 128


def _worker_id():
    return lax.axis_index("s") * NUM_CORES + lax.axis_index("c")


def _transpose_block(src_v, stg_v):
    lanes = lax.iota(jnp.int32, 16)
    parity = lanes & 1
    row_base = lanes >> 1
    col_par = parity * D_MODEL

    @plsc.parallel_loop(0, D_MODEL, unroll=4)
    def _(c):
        col = col_par + c
        for vb in range(VB // 16):
            vals = src_v[c, pl.ds(vb * 16, 16)]
            plsc.store_scatter(stg_v, [vb * 8 + row_base, col], vals)


def _tr_body(wt_hbm, tail_hbm, scratch_hbm, srcs, stgs, tail_v, isems, osems,
             *, n_full_blocks, blocks_per_w, v_tail_start, v_tail):
    wid = _worker_id()
    start = wid * blocks_per_w

    def start_in(j, p):
        pltpu.async_copy(wt_hbm.at[:, pl.ds(j * VB, VB)],
                         srcs[p].at[:, pl.ds(0, VB)], isems[p])

    def wait_in(p):
        pltpu.make_async_copy(wt_hbm.at[:, pl.ds(0, VB)],
                              srcs[p].at[:, pl.ds(0, VB)], isems[p]).wait()

    def start_out(j, p):
        pltpu.async_copy(stgs[p], scratch_hbm.at[pl.ds(j * (VB // 2), VB // 2)],
                         osems[p])

    def wait_out(p):
        pltpu.make_async_copy(stgs[p], scratch_hbm.at[pl.ds(0, VB // 2)],
                              osems[p]).wait()

    for p in range(2):
        start_in(start + p, p)

    @pl.loop(0, blocks_per_w, step=2)
    def _(k):
        for p in range(2):
            j = start + k + p
            wait_in(p)

            @pl.when(k + p >= 2)
            def _():
                wait_out(p)

            _transpose_block(srcs[p], stgs[p])
            start_out(j, p)

            @pl.when(k + p + 2 < blocks_per_w)
            def _():
                start_in(j + 2, p)

    for p in range(2):
        wait_out(p)

    n_extra = n_full_blocks - blocks_per_w * NUM_WORKERS

    @pl.when(wid < n_extra)
    def _():
        j = blocks_per_w * NUM_WORKERS + wid
        start_in(j, 0)
        wait_in(0)
        _transpose_block(srcs[0], stgs[0])
        start_out(j, 0)
        wait_out(0)

    @pl.when(wid == NUM_WORKERS - 1)
    def _():
        pltpu.sync_copy(tail_hbm, tail_v)

        @pl.loop(0, v_tail // 2)
        def _(r2):
            for half in range(2):
                for c4 in range(4):
                    stgs[1][r2, pl.ds(half * 64 + c4 * 16, 16)] = (
                        tail_v[2 * r2 + half, pl.ds(c4 * 16, 16)])

        pltpu.async_copy(stgs[1].at[pl.ds(0, v_tail // 2)],
                         scratch_hbm.at[pl.ds(v_tail_start // 2, v_tail // 2)],
                         osems[1])
        pltpu.make_async_copy(stgs[1].at[pl.ds(0, v_tail // 2)],
                              scratch_hbm.at[pl.ds(0, v_tail // 2)],
                              osems[1]).wait()


def _emb_body(idx_hbm, table_hbm, out_hbm, idx_v, stg, gsems, ssems, *,
              rows_per_w, t):
    wid = _worker_id()
    row0 = wid * rows_per_w
    base = row0 * t

    pltpu.sync_copy(idx_hbm.at[pl.ds(row0, rows_per_w)], idx_v)

    def start_gather(g, b):
        pltpu.async_copy(table_hbm.at[idx_v.at[g]], stg[b], gsems[b])

    def start_store(g, b):
        pltpu.async_copy(stg[b],
                         out_hbm.at[pl.ds(base + g * t, t), pl.ds(0, D_MODEL)],
                         ssems[b])

    def wait_gather(b):
        pltpu.make_async_copy(table_hbm.at[pl.ds(0, t)], stg[b],
                              gsems[b]).wait()

    def wait_store(b):
        pltpu.make_async_copy(stg[b],
                              out_hbm.at[pl.ds(0, t), pl.ds(0, D_MODEL)],
                              ssems[b]).wait()

    for b in range(NBUF):
        start_gather(b, b)

    @pl.loop(0, rows_per_w - NBUF, step=NBUF)
    def _(k):
        for b in range(NBUF):
            g = k + b
            wait_gather(b)
            start_store(g, b)
            wait_store(b)
            start_gather(g + NBUF, b)

    for b in range(NBUF):
        wait_gather(b)
        start_store(rows_per_w - NBUF + b, b)
    for b in range(NBUF):
        wait_store(b)


def kernel(input, weight):
    bsz, t = input.shape
    n = bsz * t
    v, d = weight.shape
    assert d == D_MODEL and bsz % NUM_WORKERS == 0
    rows_per_w = bsz // NUM_WORKERS
    n_full_blocks = v // VB
    blocks_per_w = (n_full_blocks // (2 * NUM_WORKERS)) * 2
    v_tail_start = n_full_blocks * VB
    v_tail = v - v_tail_start

    idx = input.astype(jnp.int32)
    wt = weight.T
    tail = weight[v_tail_start:]

    mesh = plsc.VectorSubcoreMesh(core_axis_name="c", subcore_axis_name="s")

    tr_body = functools.partial(
        _tr_body, n_full_blocks=n_full_blocks, blocks_per_w=blocks_per_w,
        v_tail_start=v_tail_start, v_tail=v_tail)
    scratch = pl.kernel(
        tr_body,
        out_type=jax.ShapeDtypeStruct((v // 2, D_PAD), jnp.float32),
        mesh=mesh,
        compiler_params=pltpu.CompilerParams(
            use_tc_tiling_on_sc=True,
            needs_layout_passes=False,
            skip_device_barrier=True,
            disable_bounds_checks=True,
            disable_semaphore_checks=True,
        ),
        scratch_types=[
            [pltpu.VMEM((D_MODEL, VB + 8), jnp.float32) for _ in range(2)],
            [pltpu.VMEM((VB // 2, D_PAD), jnp.float32) for _ in range(2)],
            pltpu.VMEM((v_tail, D_MODEL), jnp.float32),
            [pltpu.SemaphoreType.DMA for _ in range(2)],
            [pltpu.SemaphoreType.DMA for _ in range(2)],
        ],
    )(wt, tail)

    body = functools.partial(_emb_body, rows_per_w=rows_per_w, t=t)
    out = pl.kernel(
        body,
        out_type=jax.ShapeDtypeStruct((n, D_PAD), jnp.float32),
        mesh=mesh,
        compiler_params=pltpu.CompilerParams(
            use_tc_tiling_on_sc=False,
            skip_device_barrier=True,
            disable_bounds_checks=True,
            disable_semaphore_checks=True,
        ),
        scratch_types=[
            pltpu.VMEM((rows_per_w, t), jnp.int32),
            [pltpu.VMEM((t, D_MODEL), jnp.float32) for _ in range(NBUF)],
            [pltpu.SemaphoreType.DMA for _ in range(NBUF)],
            [pltpu.SemaphoreType.DMA for _ in range(NBUF)],
        ],
    )(idx, scratch.reshape(v, D_MODEL))
    return out[:, :D_MODEL].reshape(bsz, t, D_MODEL)


# speedup vs baseline: 1.2895x; 1.2895x over previous
import functools

import jax
import jax.numpy as jnp
from jax import lax
from jax.experimental import pallas as pl
from jax.experimental.pallas import tpu as pltpu
from jax.experimental.pallas import tpu_sc as plsc

D_MODEL = 64
D_PAD = 128
NUM_CORES = 2
NUM_SUBCORES = 16
NUM_WORKERS = NUM_CORES * NUM_SUBCORES
NBUF = 2


def _emb_body(idx_hbm, table_hbm, out_hbm, idx_v, stg, gsems, ssems, *,
              rows_per_w, t):
    c = lax.axis_index("c")
    s = lax.axis_index("s")
    wid = s * NUM_CORES + c
    row0 = wid * rows_per_w
    base = row0 * t

    pltpu.sync_copy(idx_hbm.at[pl.ds(row0, rows_per_w)], idx_v)

    def start_gather(g, b):
        pltpu.async_copy(table_hbm.at[idx_v.at[g]], stg[b], gsems[b])

    def start_store(g, b):
        pltpu.async_copy(stg[b],
                         out_hbm.at[pl.ds(base + g * t, t), pl.ds(0, D_MODEL)],
                         ssems[b])

    def wait_gather(b):
        pltpu.make_async_copy(table_hbm.at[pl.ds(0, t)], stg[b],
                              gsems[b]).wait()

    def wait_store(b):
        pltpu.make_async_copy(stg[b],
                              out_hbm.at[pl.ds(0, t), pl.ds(0, D_MODEL)],
                              ssems[b]).wait()

    for b in range(NBUF):
        start_gather(b, b)

    @pl.loop(0, rows_per_w - NBUF, step=NBUF)
    def _(k):
        for b in range(NBUF):
            g = k + b
            wait_gather(b)
            start_store(g, b)
            wait_store(b)
            start_gather(g + NBUF, b)

    for b in range(NBUF):
        wait_gather(b)
        start_store(rows_per_w - NBUF + b, b)
    for b in range(NBUF):
        wait_store(b)


def kernel(input, weight):
    bsz, t = input.shape
    n = bsz * t
    assert bsz % NUM_WORKERS == 0
    rows_per_w = bsz // NUM_WORKERS
    idx = input.astype(jnp.int32)

    body = functools.partial(_emb_body, rows_per_w=rows_per_w, t=t)
    mesh = plsc.VectorSubcoreMesh(core_axis_name="c", subcore_axis_name="s")
    out = pl.kernel(
        body,
        out_type=jax.ShapeDtypeStruct((n, D_PAD), jnp.float32),
        mesh=mesh,
        compiler_params=pltpu.CompilerParams(
            use_tc_tiling_on_sc=False,
            skip_device_barrier=True,
            disable_bounds_checks=True,
            disable_semaphore_checks=True,
        ),
        scratch_types=[
            pltpu.VMEM((rows_per_w, t), jnp.int32),
            [pltpu.VMEM((t, D_MODEL), jnp.float32) for _ in range(NBUF)],
            [pltpu.SemaphoreType.DMA for _ in range(NBUF)],
            [pltpu.SemaphoreType.DMA for _ in range(NBUF)],
        ],
    )(idx, weight)
    return out[:, :D_MODEL].reshape(bsz, t, D_MODEL)
